# baseline (speedup 1.0000x reference)
import functools

import jax
import jax.numpy as jnp
from jax import lax
from jax.experimental import pallas as pl
from jax.experimental.pallas import tpu as pltpu
from jax.experimental.pallas import tpu_sc as plsc

_NC = 2
_NS = 16
_NW = _NC * _NS
_LANES = 16

_CARD = 1000
_D_TOK = 16


def _sc_gather(cat_emb, xcat_flat, b):
    total = xcat_flat.shape[0]
    d = cat_emb.shape[1]
    per_w = total // _NW
    chunk = 80
    n_chunks = per_w // chunk
    assert n_chunks * chunk == per_w and per_w % 8 == 0

    mesh = plsc.VectorSubcoreMesh(
        core_axis_name="c", subcore_axis_name="s",
        num_cores=_NC, num_subcores=_NS,
    )

    @functools.partial(
        pl.kernel,
        out_type=jax.ShapeDtypeStruct((total, d), jnp.float32),
        mesh=mesh,
        scratch_types=[
            pltpu.VMEM((n_chunks, chunk), jnp.int32),
            pltpu.VMEM((n_chunks, chunk, d), jnp.float32),
            pltpu.SemaphoreType.DMA,
        ],
        compiler_params=pltpu.CompilerParams(use_tc_tiling_on_sc=False),
    )
    def gather_kernel(emb_hbm, idx_hbm, out_hbm, idx_v, rows_v, sem):
        wid = lax.axis_index("s") * _NC + lax.axis_index("c")
        base = wid * per_w
        for k in range(n_chunks):
            pltpu.sync_copy(idx_hbm.at[pl.ds(base + k * chunk, chunk)],
                            idx_v.at[k])
        shift = b.bit_length() - 1
        iota = lax.broadcasted_iota(jnp.int32, (_LANES,), 0)
        for k in range(n_chunks):
            for c in range(chunk // _LANES):
                row = base + k * chunk + c * _LANES + iota
                field = lax.shift_right_logical(row, shift)
                sl = pl.ds(c * _LANES, _LANES)
                idx_v[k, sl] = idx_v[k, sl] + field * _CARD
        copies = [
            pltpu.async_copy(emb_hbm.at[idx_v.at[k]], rows_v.at[k], sem)
            for k in range(n_chunks)
        ]
        for k in range(n_chunks):
            copies[k].wait()
            pltpu.sync_copy(rows_v.at[k],
                            out_hbm.at[pl.ds(base + k * chunk, chunk)])

    return gather_kernel(cat_emb, xcat_flat)


def _tc_body(n_cat, bt, n_dma, xnum_ref, g_ref, tokw_ref, tokb_ref, recw_ref,
             rlwt_ref, rlb_ref, onum_ref, ocat_hbm, buf0, buf1, sems):
    f32 = jnp.float32
    xnum = xnum_ref[...]
    d_num = xnum.shape[1]
    i = pl.program_id(0)
    nstep = pl.num_programs(0)
    card = ocat_hbm.shape[2]
    rc = bt // n_dma

    recw = recw_ref[...]
    a = jnp.sum(tokw_ref[0:d_num, :] * recw, axis=1)
    bias13 = jnp.concatenate(
        [jnp.zeros((1, _D_TOK), f32), tokb_ref[0:d_num - 1, :]], axis=0)
    c = jnp.sum(bias13 * recw, axis=1)
    rows = lax.broadcasted_iota(jnp.int32, (d_num, d_num), 0)
    cols = lax.broadcasted_iota(jnp.int32, (d_num, d_num), 1)
    m = jnp.where(cols == rows + 1, a[None, :], jnp.zeros((), f32))
    c2 = c + jnp.where(
        lax.broadcasted_iota(jnp.int32, (d_num,), 0) == 0, a[0], 0.0)
    onum_ref[...] = lax.dot_general(
        xnum, m, (((1,), (0,)), ((), ())),
        preferred_element_type=f32) + c2[None, :]

    h0 = xnum[:, d_num - 1:d_num] * tokw_ref[d_num:d_num + 1, :] \
        + tokb_ref[d_num - 1:d_num, :]

    def compute_into(buf):
        for n in range(n_cat):
            if n == 0:
                h = h0
            else:
                h = g_ref[n - 1] + tokb_ref[d_num - 1 + n:d_num + n, :]
            wt = rlwt_ref[n]
            out = lax.dot_general(h.astype(jnp.bfloat16), wt,
                                  (((1,), (0,)), ((), ())),
                                  preferred_element_type=f32)
            buf[:, n, :] = out + rlb_ref[n, :][None, :]

    def dma(buf, slot, step):
        return [
            pltpu.make_async_copy(
                buf.at[pl.ds(s * rc, rc)],
                ocat_hbm.at[pl.ds(step * bt + s * rc, rc)],
                sems.at[slot, s])
            for s in range(n_dma)
        ]

    slot = lax.rem(i, 2)
    for k, buf in ((0, buf0), (1, buf1)):
        @pl.when(slot == k)
        def _(buf=buf):
            compute_into(buf)

    for k, buf in ((0, buf0), (1, buf1)):
        @pl.when((i >= 1) & (slot == 1 - k))
        def _(k=k, buf=buf):
            for c in dma(buf, k, i - 1):
                c.wait()

    for k, buf in ((0, buf0), (1, buf1)):
        @pl.when(slot == k)
        def _(k=k, buf=buf):
            for c in dma(buf, k, i):
                c.start()

    for k, buf in ((0, buf0), (1, buf1)):
        @pl.when((i == nstep - 1) & (slot == k))
        def _(k=k, buf=buf):
            for c in dma(buf, k, i):
                c.wait()


def kernel(x_num, x_cat, tok_weight, tok_bias, cat_emb, category_offsets,
           rec_weight, rec_lin_w, rec_lin_b):
    b, d_num = x_num.shape
    n_cat = x_cat.shape[1]
    n_used = n_cat - 1
    card = rec_lin_w.shape[1]

    xcat_flat = x_cat[:, :n_used].T.reshape(n_used * b)
    g = _sc_gather(cat_emb, xcat_flat, b).reshape(n_used, b, _D_TOK)
    rlwt = rec_lin_w.transpose(0, 2, 1).astype(jnp.bfloat16)

    bt = 128
    n_dma = 4
    grid = (b // bt,)
    onum, ocat = pl.pallas_call(
        functools.partial(_tc_body, n_cat, bt, n_dma),
        grid=grid,
        in_specs=[
            pl.BlockSpec((bt, d_num), lambda i: (i, 0)),
            pl.BlockSpec((n_used, bt, _D_TOK), lambda i: (0, i, 0)),
            pl.BlockSpec(tok_weight.shape, lambda i: (0, 0)),
            pl.BlockSpec(tok_bias.shape, lambda i: (0, 0)),
            pl.BlockSpec(rec_weight.shape, lambda i: (0, 0)),
            pl.BlockSpec(rlwt.shape, lambda i: (0, 0, 0)),
            pl.BlockSpec(rec_lin_b.shape, lambda i: (0, 0)),
        ],
        out_specs=[
            pl.BlockSpec((bt, d_num), lambda i: (i, 0)),
            pl.BlockSpec(memory_space=pltpu.MemorySpace.HBM),
        ],
        out_shape=[
            jax.ShapeDtypeStruct((b, d_num), jnp.float32),
            jax.ShapeDtypeStruct((b, n_cat, card), jnp.float32),
        ],
        scratch_shapes=[
            pltpu.VMEM((bt, n_cat, card), jnp.float32),
            pltpu.VMEM((bt, n_cat, card), jnp.float32),
            pltpu.SemaphoreType.DMA((2, n_dma)),
        ],
    )(x_num, g, tok_weight, tok_bias, rec_weight, rlwt, rec_lin_b)
    return onum, ocat


# speedup vs baseline: 2.3227x; 2.3227x over previous
import functools

import jax
import jax.numpy as jnp
from jax import lax
from jax.experimental import pallas as pl
from jax.experimental.pallas import tpu as pltpu
from jax.experimental.pallas import tpu_sc as plsc

_NC = 2
_NS = 16
_NW = _NC * _NS
_LANES = 16

_CARD = 1000
_D_TOK = 16


def _sc_gather(cat_emb, xcat_flat, b):
    total = xcat_flat.shape[0]
    d = cat_emb.shape[1]
    per_w = total // _NW
    chunk = 80
    n_chunks = per_w // chunk
    assert n_chunks * chunk == per_w and per_w % 8 == 0

    mesh = plsc.VectorSubcoreMesh(
        core_axis_name="c", subcore_axis_name="s",
        num_cores=_NC, num_subcores=_NS,
    )

    @functools.partial(
        pl.kernel,
        out_type=jax.ShapeDtypeStruct((total, d), jnp.float32),
        mesh=mesh,
        scratch_types=[
            pltpu.VMEM((n_chunks, chunk), jnp.int32),
            pltpu.VMEM((n_chunks, chunk, d), jnp.float32),
            pltpu.SemaphoreType.DMA,
        ],
        compiler_params=pltpu.CompilerParams(use_tc_tiling_on_sc=False),
    )
    def gather_kernel(emb_hbm, idx_hbm, out_hbm, idx_v, rows_v, sem):
        wid = lax.axis_index("s") * _NC + lax.axis_index("c")
        base = wid * per_w
        for k in range(n_chunks):
            pltpu.sync_copy(idx_hbm.at[pl.ds(base + k * chunk, chunk)],
                            idx_v.at[k])
        shift = b.bit_length() - 1
        iota = lax.broadcasted_iota(jnp.int32, (_LANES,), 0)
        for k in range(n_chunks):
            for c in range(chunk // _LANES):
                row = base + k * chunk + c * _LANES + iota
                field = lax.shift_right_logical(row, shift)
                sl = pl.ds(c * _LANES, _LANES)
                idx_v[k, sl] = idx_v[k, sl] + field * _CARD
        copies = [
            pltpu.async_copy(emb_hbm.at[idx_v.at[k]], rows_v.at[k], sem)
            for k in range(n_chunks)
        ]
        for k in range(n_chunks):
            copies[k].wait()
            pltpu.sync_copy(rows_v.at[k],
                            out_hbm.at[pl.ds(base + k * chunk, chunk)])

    return gather_kernel(cat_emb, xcat_flat)


def _tc_body(n_cat, xnumt_ref, g_ref, tokw_ref, tokb_ref, recw_ref,
             rlw_ref, rlb_ref, onumt_ref, ocat_ref):
    f32 = jnp.float32
    bf16 = jnp.bfloat16
    d_num = xnumt_ref.shape[0]
    n = pl.program_id(0)

    @pl.when(n == 0)
    def _():
        recw = recw_ref[...]
        a = jnp.sum(tokw_ref[0:d_num, :] * recw, axis=1)
        bias13 = jnp.concatenate(
            [jnp.zeros((1, _D_TOK), f32), tokb_ref[0:d_num - 1, :]], axis=0)
        c = jnp.sum(bias13 * recw, axis=1)
        rows = lax.broadcasted_iota(jnp.int32, (d_num, d_num), 0)
        cols = lax.broadcasted_iota(jnp.int32, (d_num, d_num), 1)
        m2 = jnp.where(cols == rows - 1, a[:, None], jnp.zeros((), f32))
        c2 = c + jnp.where(
            lax.broadcasted_iota(jnp.int32, (d_num,), 0) == 0, a[0], 0.0)
        onumt_ref[...] = lax.dot_general(
            m2, xnumt_ref[...], (((1,), (0,)), ((), ())),
            preferred_element_type=f32) + c2[:, None]

    w = rlw_ref[0].astype(bf16)
    rlb_col = rlb_ref[0, 0, :]

    @pl.when(n == 0)
    def _():
        h0t = tokw_ref[d_num, :][:, None] \
            * xnumt_ref[d_num - 1:d_num, :] \
            + tokb_ref[d_num - 1, :][:, None]
        out = lax.dot_general(w, h0t.astype(bf16), (((1,), (0,)), ((), ())),
                              preferred_element_type=f32)
        ocat_ref[0] = out + rlb_col[:, None]

    @pl.when(n > 0)
    def _():
        h = g_ref[n - 1] + tokb_ref[pl.ds(d_num - 1 + n, 1), :]
        out = lax.dot_general(w, h.astype(bf16), (((1,), (1,)), ((), ())),
                              preferred_element_type=f32)
        ocat_ref[0] = out + rlb_col[:, None]


def kernel(x_num, x_cat, tok_weight, tok_bias, cat_emb, category_offsets,
           rec_weight, rec_lin_w, rec_lin_b):
    b, d_num = x_num.shape
    n_cat = x_cat.shape[1]
    n_used = n_cat - 1
    card = rec_lin_w.shape[1]

    xcat_flat = x_cat[:, :n_used].T.reshape(n_used * b)
    g = _sc_gather(cat_emb, xcat_flat, b).reshape(n_used, b, _D_TOK)
    xnumt = x_num.T
    rlb3 = rec_lin_b.reshape(n_cat, 1, card)

    grid = (n_cat,)
    onumt, ocat_p = pl.pallas_call(
        functools.partial(_tc_body, n_cat),
        grid=grid,
        in_specs=[
            pl.BlockSpec(xnumt.shape, lambda n: (0, 0)),
            pl.BlockSpec(g.shape, lambda n: (0, 0, 0)),
            pl.BlockSpec(tok_weight.shape, lambda n: (0, 0)),
            pl.BlockSpec(tok_bias.shape, lambda n: (0, 0)),
            pl.BlockSpec(rec_weight.shape, lambda n: (0, 0)),
            pl.BlockSpec((1, card, _D_TOK), lambda n: (n, 0, 0)),
            pl.BlockSpec((1, 1, card), lambda n: (n, 0, 0)),
        ],
        out_specs=[
            pl.BlockSpec(xnumt.shape, lambda n: (0, 0)),
            pl.BlockSpec((1, card, b), lambda n: (n, 0, 0)),
        ],
        out_shape=[
            jax.ShapeDtypeStruct((d_num, b), jnp.float32),
            jax.ShapeDtypeStruct((n_cat, card, b), jnp.float32),
        ],
    )(xnumt, g, tok_weight, tok_bias, rec_weight, rec_lin_w, rlb3)
    return onumt.T, ocat_p.transpose(2, 0, 1)


# speedup vs baseline: 2.6067x; 1.1223x over previous
import functools

import jax
import jax.numpy as jnp
from jax import lax
from jax.experimental import pallas as pl
from jax.experimental.pallas import tpu as pltpu
from jax.experimental.pallas import tpu_sc as plsc

_NC = 2
_NS = 16
_NW = _NC * _NS
_LANES = 16

_CARD = 1000
_D_TOK = 16


def _sc_gather(cat_emb, xcat_flat, b):
    total = xcat_flat.shape[0]
    d = cat_emb.shape[1]
    dt = cat_emb.dtype
    per_w = total // _NW
    chunk = 80
    n_chunks = per_w // chunk
    assert n_chunks * chunk == per_w and per_w % 8 == 0

    mesh = plsc.VectorSubcoreMesh(
        core_axis_name="c", subcore_axis_name="s",
        num_cores=_NC, num_subcores=_NS,
    )

    @functools.partial(
        pl.kernel,
        out_type=jax.ShapeDtypeStruct((total, d), dt),
        mesh=mesh,
        scratch_types=[
            pltpu.VMEM((n_chunks, chunk), jnp.int32),
            pltpu.VMEM((n_chunks, chunk, d), dt),
            pltpu.SemaphoreType.DMA,
        ],
        compiler_params=pltpu.CompilerParams(use_tc_tiling_on_sc=False),
    )
    def gather_kernel(emb_hbm, idx_hbm, out_hbm, idx_v, rows_v, sem):
        wid = lax.axis_index("s") * _NC + lax.axis_index("c")
        base = wid * per_w
        for k in range(n_chunks):
            pltpu.sync_copy(idx_hbm.at[pl.ds(base + k * chunk, chunk)],
                            idx_v.at[k])
        shift = b.bit_length() - 1
        iota = lax.broadcasted_iota(jnp.int32, (_LANES,), 0)
        for k in range(n_chunks):
            for c in range(chunk // _LANES):
                row = base + k * chunk + c * _LANES + iota
                field = lax.shift_right_logical(row, shift)
                sl = pl.ds(c * _LANES, _LANES)
                idx_v[k, sl] = idx_v[k, sl] + field * _CARD
        copies = [
            pltpu.async_copy(emb_hbm.at[idx_v.at[k]], rows_v.at[k], sem)
            for k in range(n_chunks)
        ]
        for k in range(n_chunks):
            copies[k].wait()
            pltpu.sync_copy(rows_v.at[k],
                            out_hbm.at[pl.ds(base + k * chunk, chunk)])

    return gather_kernel(cat_emb, xcat_flat)


def _tc_body(n_cat, xnumt_ref, g_ref, tokw_ref, tokb_ref, recw_ref,
             rlw_ref, rlb_ref, onumt_ref, ocat_ref):
    f32 = jnp.float32
    bf16 = jnp.bfloat16
    d_num = xnumt_ref.shape[0]
    n = pl.program_id(0)

    @pl.when(n == 0)
    def _():
        recw = recw_ref[...]
        a = jnp.sum(tokw_ref[0:d_num, :] * recw, axis=1)
        bias13 = jnp.concatenate(
            [jnp.zeros((1, _D_TOK), f32), tokb_ref[0:d_num - 1, :]], axis=0)
        c = jnp.sum(bias13 * recw, axis=1)
        rows = lax.broadcasted_iota(jnp.int32, (d_num, d_num), 0)
        cols = lax.broadcasted_iota(jnp.int32, (d_num, d_num), 1)
        m2 = jnp.where(cols == rows - 1, a[:, None], jnp.zeros((), f32))
        c2 = c + jnp.where(
            lax.broadcasted_iota(jnp.int32, (d_num,), 0) == 0, a[0], 0.0)
        onumt_ref[...] = lax.dot_general(
            m2, xnumt_ref[...], (((1,), (0,)), ((), ())),
            preferred_element_type=f32) + c2[:, None]

    wt = rlw_ref[0].astype(bf16)
    rlb_col = rlb_ref[0, 0, :]

    @pl.when(n == 0)
    def _():
        h0t = tokw_ref[d_num, :][:, None] \
            * xnumt_ref[d_num - 1:d_num, :] \
            + tokb_ref[d_num - 1, :][:, None]
        out = lax.dot_general(wt, h0t.astype(bf16), (((0,), (0,)), ((), ())),
                              preferred_element_type=f32)
        ocat_ref[0] = out + rlb_col[:, None]

    @pl.when(n > 0)
    def _():
        h = g_ref[n - 1] \
            + tokb_ref[pl.ds(d_num - 1 + n, 1), :].astype(bf16)
        out = lax.dot_general(wt, h, (((0,), (1,)), ((), ())),
                              preferred_element_type=f32)
        ocat_ref[0] = out + rlb_col[:, None]


def kernel(x_num, x_cat, tok_weight, tok_bias, cat_emb, category_offsets,
           rec_weight, rec_lin_w, rec_lin_b):
    b, d_num = x_num.shape
    n_cat = x_cat.shape[1]
    n_used = n_cat - 1
    card = rec_lin_w.shape[1]

    xcat_flat = x_cat[:, :n_used].T.reshape(n_used * b)
    emb_bf = cat_emb.astype(jnp.bfloat16)
    g = _sc_gather(emb_bf, xcat_flat, b).reshape(n_used, b, _D_TOK)
    xnumt = x_num.T
    rlwt = rec_lin_w.transpose(0, 2, 1)
    rlb3 = rec_lin_b.reshape(n_cat, 1, card)

    grid = (n_cat,)
    onumt, ocat_p = pl.pallas_call(
        functools.partial(_tc_body, n_cat),
        grid=grid,
        in_specs=[
            pl.BlockSpec(xnumt.shape, lambda n: (0, 0)),
            pl.BlockSpec(g.shape, lambda n: (0, 0, 0)),
            pl.BlockSpec(tok_weight.shape, lambda n: (0, 0)),
            pl.BlockSpec(tok_bias.shape, lambda n: (0, 0)),
            pl.BlockSpec(rec_weight.shape, lambda n: (0, 0)),
            pl.BlockSpec((1, _D_TOK, card), lambda n: (n, 0, 0)),
            pl.BlockSpec((1, 1, card), lambda n: (n, 0, 0)),
        ],
        out_specs=[
            pl.BlockSpec(xnumt.shape, lambda n: (0, 0)),
            pl.BlockSpec((1, card, b), lambda n: (n, 0, 0)),
        ],
        out_shape=[
            jax.ShapeDtypeStruct((d_num, b), jnp.float32),
            jax.ShapeDtypeStruct((n_cat, card, b), jnp.float32),
        ],
    )(xnumt, g, tok_weight, tok_bias, rec_weight, rlwt, rlb3)
    return onumt.T, ocat_p.transpose(2, 0, 1)


# speedup vs baseline: 3.2098x; 1.2314x over previous
import functools

import jax
import jax.numpy as jnp
from jax import lax
from jax.experimental import pallas as pl
from jax.experimental.pallas import tpu as pltpu
from jax.experimental.pallas import tpu_sc as plsc

_NC = 2
_NS = 16
_NW = _NC * _NS
_LANES = 16

_CARD = 1000
_D_TOK = 16


def _sc_gather(emb_t, xcat_flat, b):
    d, v = emb_t.shape
    total = xcat_flat.shape[0]
    halves = _NW // d
    seg = total // halves
    shift = b.bit_length() - 1
    assert halves * d == _NW and seg * halves == total and b == (1 << shift)

    mesh = plsc.VectorSubcoreMesh(
        core_axis_name="c", subcore_axis_name="s",
        num_cores=_NC, num_subcores=_NS,
    )

    @functools.partial(
        pl.kernel,
        out_type=jax.ShapeDtypeStruct((d, total), jnp.float32),
        mesh=mesh,
        scratch_types=[
            pltpu.VMEM((v,), jnp.float32),
            pltpu.VMEM((seg,), jnp.int32),
            pltpu.VMEM((seg,), jnp.float32),
        ],
        compiler_params=pltpu.CompilerParams(use_tc_tiling_on_sc=False,
                                             needs_layout_passes=False),
    )
    def gather_kernel(emb_hbm, idx_hbm, out_hbm, plane_v, idx_v, res_v):
        wid = lax.axis_index("s") * _NC + lax.axis_index("c")
        p = lax.rem(wid, d)
        base = (wid // d) * seg
        pltpu.sync_copy(emb_hbm.at[p], plane_v)
        pltpu.sync_copy(idx_hbm.at[pl.ds(base, seg)], idx_v)
        iota = lax.broadcasted_iota(jnp.int32, (_LANES,), 0)

        def body(i, carry):
            off = i * _LANES
            field = lax.shift_right_logical(base + off + iota, shift)
            gidx = idx_v[pl.ds(off, _LANES)] + field * _CARD
            res_v[pl.ds(off, _LANES)] = plsc.load_gather(plane_v, [gidx])
            return carry

        lax.fori_loop(0, seg // _LANES, body, 0)
        pltpu.sync_copy(res_v, out_hbm.at[p, pl.ds(base, seg)])

    return gather_kernel(emb_t, xcat_flat)


def _tc_body(n_cat, xnumt_ref, g_ref, tokw_ref, tokb_ref, recw_ref,
             rlw_ref, rlb_ref, onumt_ref, ocat_ref):
    f32 = jnp.float32
    bf16 = jnp.bfloat16
    d_num = xnumt_ref.shape[0]
    n = pl.program_id(0)

    @pl.when(n == 0)
    def _():
        recw = recw_ref[...]
        a = jnp.sum(tokw_ref[0:d_num, :] * recw, axis=1)
        bias13 = jnp.concatenate(
            [jnp.zeros((1, _D_TOK), f32), tokb_ref[0:d_num - 1, :]], axis=0)
        c = jnp.sum(bias13 * recw, axis=1)
        rows = lax.broadcasted_iota(jnp.int32, (d_num, d_num), 0)
        cols = lax.broadcasted_iota(jnp.int32, (d_num, d_num), 1)
        m2 = jnp.where(cols == rows - 1, a[:, None], jnp.zeros((), f32))
        c2 = c + jnp.where(
            lax.broadcasted_iota(jnp.int32, (d_num,), 0) == 0, a[0], 0.0)
        onumt_ref[...] = lax.dot_general(
            m2, xnumt_ref[...], (((1,), (0,)), ((), ())),
            preferred_element_type=f32) + c2[:, None]

    wt = rlw_ref[0].astype(bf16)
    rlb_col = rlb_ref[0, 0, :]
    tb = tokb_ref[d_num - 1 + n, :]
    h0t = tokw_ref[d_num, :][:, None] * xnumt_ref[d_num - 1:d_num, :]
    ht = jnp.where(n == 0, h0t, g_ref[...]) + tb[:, None]
    out = lax.dot_general(wt, ht.astype(bf16), (((0,), (0,)), ((), ())),
                          preferred_element_type=f32)
    ocat_ref[0] = out + rlb_col[:, None]


def kernel(x_num, x_cat, tok_weight, tok_bias, cat_emb, category_offsets,
           rec_weight, rec_lin_w, rec_lin_b):
    b, d_num = x_num.shape
    n_cat = x_cat.shape[1]
    n_used = n_cat - 1
    card = rec_lin_w.shape[1]

    xcat_flat = x_cat[:, :n_used].T.reshape(n_used * b)
    g2 = _sc_gather(cat_emb.T, xcat_flat, b)
    xnumt = x_num.T
    rlwt = rec_lin_w.transpose(0, 2, 1)
    rlb3 = rec_lin_b.reshape(n_cat, 1, card)

    grid = (n_cat,)
    onumt, ocat_p = pl.pallas_call(
        functools.partial(_tc_body, n_cat),
        grid=grid,
        in_specs=[
            pl.BlockSpec(xnumt.shape, lambda n: (0, 0)),
            pl.BlockSpec((_D_TOK, b), lambda n: (0, jnp.maximum(n - 1, 0))),
            pl.BlockSpec(tok_weight.shape, lambda n: (0, 0)),
            pl.BlockSpec(tok_bias.shape, lambda n: (0, 0)),
            pl.BlockSpec(rec_weight.shape, lambda n: (0, 0)),
            pl.BlockSpec((1, _D_TOK, card), lambda n: (n, 0, 0)),
            pl.BlockSpec((1, 1, card), lambda n: (n, 0, 0)),
        ],
        out_specs=[
            pl.BlockSpec(xnumt.shape, lambda n: (0, 0)),
            pl.BlockSpec((1, card, b), lambda n: (n, 0, 0)),
        ],
        out_shape=[
            jax.ShapeDtypeStruct((d_num, b), jnp.float32),
            jax.ShapeDtypeStruct((n_cat, card, b), jnp.float32),
        ],
    )(xnumt, g2, tok_weight, tok_bias, rec_weight, rlwt, rlb3)
    return onumt.T, ocat_p.transpose(2, 0, 1)
